# Initial kernel scaffold; baseline (speedup 1.0000x reference)
import jax
import jax.numpy as jnp
from jax.experimental import pallas as pl


def kernel(seconds_starts_totals, start_table, total_table):
    raise NotImplementedError("write your pallas kernel here")


# speedup vs baseline: 2.8119x; 2.8119x over previous
import functools

import jax
import jax.numpy as jnp
from jax import lax
from jax.experimental import pallas as pl
from jax.experimental.pallas import tpu as pltpu
from jax.experimental.pallas import tpu_sc as plsc

_BATCH = 16384
_DIM = 128
_NC = 2
_NS = 16
_NW = _NC * _NS
_BPW = _BATCH // _NW
_CHUNK = 128
_NCHUNK = _BPW // _CHUNK


def _tec_body(starts_hbm, totals_hbm, start_hbm, total_hbm,
              out_s_hbm, out_t_hbm, idx_v, rows_v, sem):
    wid = lax.axis_index("s") * _NC + lax.axis_index("c")
    base = wid * _BPW
    for c in range(_NCHUNK):
        pltpu.sync_copy(starts_hbm.at[pl.ds(base + c * _CHUNK, _CHUNK)],
                        idx_v.at[c])
        pltpu.sync_copy(totals_hbm.at[pl.ds(base + c * _CHUNK, _CHUNK)],
                        idx_v.at[_NCHUNK + c])
    maxi = start_hbm.shape[0] - 1
    for r in range(2 * _NCHUNK):
        for off in range(0, _CHUNK, 16):
            idx_v[r, pl.ds(off, 16)] = jnp.clip(idx_v[r, pl.ds(off, 16)], 0, maxi)
    for j, (tab, out) in enumerate(((start_hbm, out_s_hbm), (total_hbm, out_t_hbm))):
        for c in range(_NCHUNK):
            pltpu.async_copy(tab.at[idx_v.at[j * _NCHUNK + c]], rows_v, sem).wait()
            pltpu.sync_copy(rows_v, out.at[pl.ds(base + c * _CHUNK, _CHUNK)])


_lookup = functools.partial(
    pl.kernel,
    out_type=(jax.ShapeDtypeStruct((_BATCH, _DIM), jnp.float32),
              jax.ShapeDtypeStruct((_BATCH, _DIM), jnp.float32)),
    mesh=plsc.VectorSubcoreMesh(core_axis_name="c", subcore_axis_name="s"),
    scratch_types=[
        pltpu.VMEM((2 * _NCHUNK, _CHUNK), jnp.int32),
        pltpu.VMEM((_CHUNK, _DIM), jnp.float32),
        pltpu.SemaphoreType.DMA,
    ],
)(_tec_body)


def kernel(seconds_starts_totals, start_table, total_table):
    out_s, out_t = _lookup(seconds_starts_totals[:, 0],
                           seconds_starts_totals[:, 1],
                           start_table, total_table)
    return (out_s[:, None, :], out_t[:, None, :])


# speedup vs baseline: 3.3297x; 1.1841x over previous
import functools

import jax
import jax.numpy as jnp
from jax import lax
from jax.experimental import pallas as pl
from jax.experimental.pallas import tpu as pltpu
from jax.experimental.pallas import tpu_sc as plsc

_BATCH = 16384
_DIM = 128
_NC = 2
_NS = 16
_NW = _NC * _NS
_BPW = _BATCH // _NW
_CHUNK = 128
_NCHUNK = _BPW // _CHUNK
_NT = 2 * _NCHUNK
_NBUF = _NT - 1


def _tec_body(starts_hbm, totals_hbm, start_hbm, total_hbm,
              out_s_hbm, out_t_hbm, idx_v, rows_v, sem_i, sem_g, sem_o):
    wid = lax.axis_index("s") * _NC + lax.axis_index("c")
    base = wid * _BPW
    idx_cps = []
    for c in range(_NCHUNK):
        idx_cps.append(pltpu.async_copy(
            starts_hbm.at[pl.ds(base + c * _CHUNK, _CHUNK)], idx_v.at[c], sem_i))
        idx_cps.append(pltpu.async_copy(
            totals_hbm.at[pl.ds(base + c * _CHUNK, _CHUNK)],
            idx_v.at[_NCHUNK + c], sem_i))
    for cp in idx_cps:
        cp.wait()
    maxi = start_hbm.shape[0] - 1
    for r in range(_NT):
        for off in range(0, _CHUNK, 16):
            idx_v[r, pl.ds(off, 16)] = jnp.clip(idx_v[r, pl.ds(off, 16)], 0, maxi)

    tabs = ((start_hbm, out_s_hbm), (total_hbm, out_t_hbm))
    gathers = [None] * _NT
    outs = [None] * _NT

    for k in range(_NBUF):
        tab = tabs[k // _NCHUNK][0]
        gathers[k] = pltpu.async_copy(
            tab.at[idx_v.at[k]], rows_v.at[k], sem_g.at[k])
    for k in range(_NT):
        if k == _NBUF:
            outs[0].wait()
            tab = tabs[k // _NCHUNK][0]
            gathers[k] = pltpu.async_copy(
                tab.at[idx_v.at[k]], rows_v.at[0], sem_g.at[k])
        buf = k % _NBUF
        tab, out = tabs[k // _NCHUNK]
        gathers[k].wait()
        c = k % _NCHUNK
        outs[k] = pltpu.async_copy(
            rows_v.at[buf], out.at[pl.ds(base + c * _CHUNK, _CHUNK)], sem_o.at[k])
    for k in range(1, _NT):
        outs[k].wait()


_lookup = functools.partial(
    pl.kernel,
    out_type=(jax.ShapeDtypeStruct((_BATCH, _DIM), jnp.float32),
              jax.ShapeDtypeStruct((_BATCH, _DIM), jnp.float32)),
    mesh=plsc.VectorSubcoreMesh(core_axis_name="c", subcore_axis_name="s"),
    scratch_types=[
        pltpu.VMEM((_NT, _CHUNK), jnp.int32),
        pltpu.VMEM((_NBUF, _CHUNK, _DIM), jnp.float32),
        pltpu.SemaphoreType.DMA,
        pltpu.SemaphoreType.DMA((_NT,)),
        pltpu.SemaphoreType.DMA((_NT,)),
    ],
)(_tec_body)


def kernel(seconds_starts_totals, start_table, total_table):
    out_s, out_t = _lookup(seconds_starts_totals[:, 0],
                           seconds_starts_totals[:, 1],
                           start_table, total_table)
    return (out_s[:, None, :], out_t[:, None, :])


# speedup vs baseline: 4.7422x; 1.4242x over previous
import functools

import jax
import jax.numpy as jnp
from jax import lax
from jax.experimental import pallas as pl
from jax.experimental.pallas import tpu as pltpu
from jax.experimental.pallas import tpu_sc as plsc

_BATCH = 16384
_DIM = 128
_NC = 2
_NS = 16
_NW = _NC * _NS
_BPW = _BATCH // _NW
_CHUNK = 128
_NCHUNK = _BPW // _CHUNK
_NT = 2 * _NCHUNK
_NBUF = _NT - 1
_ROWS = 513
_OFF_T = 520


def _tec_body(starts_hbm, totals_hbm, start_hbm, total_hbm,
              out_s_hbm, out_t_hbm, idx_v, rows_v, tabs_sh, sem_i, sem_g, sem_o):
    cid = lax.axis_index("c")
    sid = lax.axis_index("s")
    wid = sid * _NC + cid
    base = wid * _BPW
    idx_cps = []
    for c in range(_NCHUNK):
        idx_cps.append(pltpu.async_copy(
            starts_hbm.at[pl.ds(base + c * _CHUNK, _CHUNK)], idx_v.at[c], sem_i))
        idx_cps.append(pltpu.async_copy(
            totals_hbm.at[pl.ds(base + c * _CHUNK, _CHUNK)],
            idx_v.at[_NCHUNK + c], sem_i))

    @pl.when(sid == 0)
    def _():
        pltpu.sync_copy(start_hbm, tabs_sh.at[pl.ds(0, _ROWS)])

    @pl.when(sid == 1)
    def _():
        pltpu.sync_copy(total_hbm, tabs_sh.at[pl.ds(_OFF_T, _ROWS)])

    for cp in idx_cps:
        cp.wait()
    maxi = _ROWS - 1
    for r in range(_NT):
        off_r = 0 if r < _NCHUNK else _OFF_T
        for off in range(0, _CHUNK, 16):
            idx_v[r, pl.ds(off, 16)] = (
                jnp.clip(idx_v[r, pl.ds(off, 16)], 0, maxi) + off_r)
    plsc.subcore_barrier()

    gathers = [None] * _NT
    outs = [None] * _NT
    for k in range(_NBUF):
        gathers[k] = pltpu.async_copy(
            tabs_sh.at[idx_v.at[k]], rows_v.at[k], sem_g.at[k])
    for k in range(_NT):
        if k == _NBUF:
            outs[0].wait()
            gathers[k] = pltpu.async_copy(
                tabs_sh.at[idx_v.at[k]], rows_v.at[0], sem_g.at[k])
        buf = k % _NBUF
        out = out_s_hbm if k < _NCHUNK else out_t_hbm
        gathers[k].wait()
        c = k % _NCHUNK
        outs[k] = pltpu.async_copy(
            rows_v.at[buf], out.at[pl.ds(base + c * _CHUNK, _CHUNK)], sem_o.at[k])
    for k in range(1, _NT):
        outs[k].wait()


_lookup = functools.partial(
    pl.kernel,
    out_type=(jax.ShapeDtypeStruct((_BATCH, _DIM), jnp.float32),
              jax.ShapeDtypeStruct((_BATCH, _DIM), jnp.float32)),
    mesh=plsc.VectorSubcoreMesh(core_axis_name="c", subcore_axis_name="s"),
    scratch_types=[
        pltpu.VMEM((_NT, _CHUNK), jnp.int32),
        pltpu.VMEM((_NBUF, _CHUNK, _DIM), jnp.float32),
        pltpu.VMEM_SHARED((_OFF_T + _ROWS, _DIM), jnp.float32),
        pltpu.SemaphoreType.DMA,
        pltpu.SemaphoreType.DMA((_NT,)),
        pltpu.SemaphoreType.DMA((_NT,)),
    ],
)(_tec_body)


def kernel(seconds_starts_totals, start_table, total_table):
    out_s, out_t = _lookup(seconds_starts_totals[:, 0],
                           seconds_starts_totals[:, 1],
                           start_table, total_table)
    return (out_s[:, None, :], out_t[:, None, :])


# speedup vs baseline: 4.7941x; 1.0110x over previous
import functools

import jax
import jax.numpy as jnp
from jax import lax
from jax.experimental import pallas as pl
from jax.experimental.pallas import tpu as pltpu
from jax.experimental.pallas import tpu_sc as plsc

_BATCH = 16384
_DIM = 128
_NC = 2
_NS = 16
_NW = _NC * _NS
_BPW = _BATCH // _NW
_CHUNK = 128
_NCHUNK = _BPW // _CHUNK
_NT = 2 * _NCHUNK
_NBUF = _NT - 1
_ROWS = 513
_OFF_T = 520


def _tec_body(starts_hbm, totals_hbm, start_hbm, total_hbm,
              out_s_hbm, out_t_hbm, idx_v, rows_v, tabs_sh, sem_i, sem_g, sem_o):
    cid = lax.axis_index("c")
    sid = lax.axis_index("s")
    wid = sid * _NC + cid
    base = wid * _BPW
    idx_cps = []
    for c in range(_NCHUNK):
        idx_cps.append(pltpu.async_copy(
            starts_hbm.at[pl.ds(base + c * _CHUNK, _CHUNK)], idx_v.at[c], sem_i))
        idx_cps.append(pltpu.async_copy(
            totals_hbm.at[pl.ds(base + c * _CHUNK, _CHUNK)],
            idx_v.at[_NCHUNK + c], sem_i))

    @pl.when(sid == 0)
    def _():
        pltpu.sync_copy(start_hbm, tabs_sh.at[pl.ds(0, _ROWS)])

    @pl.when(sid == 1)
    def _():
        pltpu.sync_copy(total_hbm, tabs_sh.at[pl.ds(_OFF_T, _ROWS)])

    for cp in idx_cps:
        cp.wait()
    plsc.subcore_barrier()

    gathers = [None] * _NT
    outs = [None] * _NT
    for k in range(_NBUF):
        gathers[k] = pltpu.async_copy(
            tabs_sh.at[idx_v.at[k]], rows_v.at[k], sem_g.at[k])
    for k in range(_NT):
        if k == _NBUF:
            outs[0].wait()
            gathers[k] = pltpu.async_copy(
                tabs_sh.at[idx_v.at[k]], rows_v.at[0], sem_g.at[k])
        buf = k % _NBUF
        out = out_s_hbm if k < _NCHUNK else out_t_hbm
        gathers[k].wait()
        c = k % _NCHUNK
        outs[k] = pltpu.async_copy(
            rows_v.at[buf], out.at[pl.ds(base + c * _CHUNK, _CHUNK)], sem_o.at[k])
    for k in range(1, _NT):
        outs[k].wait()


_lookup = functools.partial(
    pl.kernel,
    out_type=(jax.ShapeDtypeStruct((_BATCH, _DIM), jnp.float32),
              jax.ShapeDtypeStruct((_BATCH, _DIM), jnp.float32)),
    mesh=plsc.VectorSubcoreMesh(core_axis_name="c", subcore_axis_name="s"),
    scratch_types=[
        pltpu.VMEM((_NT, _CHUNK), jnp.int32),
        pltpu.VMEM((_NBUF, _CHUNK, _DIM), jnp.float32),
        pltpu.VMEM_SHARED((_OFF_T + _ROWS, _DIM), jnp.float32),
        pltpu.SemaphoreType.DMA,
        pltpu.SemaphoreType.DMA((_NT,)),
        pltpu.SemaphoreType.DMA((_NT,)),
    ],
)(_tec_body)


def kernel(seconds_starts_totals, start_table, total_table):
    maxi = start_table.shape[0] - 1
    sst = jnp.clip(seconds_starts_totals, 0, maxi)
    out_s, out_t = _lookup(sst[:, 0], sst[:, 1] + _OFF_T,
                           start_table, total_table)
    return (out_s[:, None, :], out_t[:, None, :])
